# baseline (speedup 1.0000x reference)
import functools

import jax
import jax.numpy as jnp
from jax import lax
from jax.experimental import pallas as pl
from jax.experimental.pallas import tpu as pltpu
from jax.experimental.pallas import tpu_sc as plsc

_BATCH = 16384
_HIST = 50
_DIM = 32
_VOCAB_PAD = 1 << 20
_NUM_IDX = _BATCH * _HIST
_NUM_WORKERS = 32
_CHUNK = 1024
_BLOCKS_PER_T = _BATCH // _CHUNK
_NUM_UNITS = _HIST * _BLOCKS_PER_T
_UNITS_PER_WORKER = _NUM_UNITS // _NUM_WORKERS
_NBUF = 3

_PACK_B = 2048
_PACK_Q = _PACK_B // 4
_PACK_GRID = 489

_UNPACK_Q = _CHUNK // 4


def _pack_table(tt):
    def body(tt_ref, o_ref):
        t = tt_ref[...].T
        for s in range(4):
            o_ref[:, 32 * s : 32 * (s + 1)] = t[_PACK_Q * s : _PACK_Q * (s + 1), :]

    return pl.pallas_call(
        body,
        grid=(_PACK_GRID,),
        in_specs=[pl.BlockSpec((_DIM, _PACK_B), lambda i: (0, i))],
        out_specs=pl.BlockSpec((_PACK_Q, 128), lambda i: (i, 0)),
        out_shape=jax.ShapeDtypeStruct((_VOCAB_PAD // 4, 128), jnp.float32),
    )(tt)


def _unpack_out(packed):
    def body(in_ref, o_ref):
        rows = in_ref[...]
        for s in range(4):
            o_ref[0, :, _UNPACK_Q * s : _UNPACK_Q * (s + 1)] = rows[
                :, 32 * s : 32 * (s + 1)
            ].T

    return pl.pallas_call(
        body,
        grid=(_HIST, _BLOCKS_PER_T),
        in_specs=[
            pl.BlockSpec(
                (_CHUNK // 4, 128),
                lambda t, j: (t * _BLOCKS_PER_T + j, 0),
            )
        ],
        out_specs=pl.BlockSpec((1, _DIM, _CHUNK), lambda t, j: (t, 0, j)),
        out_shape=jax.ShapeDtypeStruct((_HIST, _DIM, _BATCH), jnp.float32),
    )(packed)


def kernel(x, table):
    xt = x.T
    xt_p = (
        xt.reshape(_HIST, _BLOCKS_PER_T, 4, _UNPACK_Q)
        .transpose(0, 1, 3, 2)
        .reshape(_HIST, _BATCH)
    )
    idx2 = (xt_p & -2048) | ((xt_p & 511) << 2) | ((xt_p >> 9) & 3)

    rt = _pack_table(table.T).reshape(_VOCAB_PAD, _DIM)
    mesh = plsc.VectorSubcoreMesh(core_axis_name="c", subcore_axis_name="s")

    @functools.partial(
        pl.kernel,
        mesh=mesh,
        out_type=jax.ShapeDtypeStruct((_NUM_IDX, _DIM), jnp.float32),
        compiler_params=pltpu.CompilerParams(use_tc_tiling_on_sc=False),
        scratch_types=(
            [pltpu.VMEM((_CHUNK,), jnp.int32)] * _NBUF
            + [pltpu.VMEM((_CHUNK, _DIM), jnp.float32)] * _NBUF
            + [pltpu.SemaphoreType.DMA] * (2 * _NBUF)
        ),
    )
    def gather_kernel(table_hbm, idx_hbm, out_hbm, *scratch):
        idx_v = scratch[:_NBUF]
        rows = scratch[_NBUF : 2 * _NBUF]
        gsem = scratch[2 * _NBUF : 3 * _NBUF]
        osem = scratch[3 * _NBUF :]
        wid = lax.axis_index("s") * 2 + lax.axis_index("c")
        u0 = wid * _UNITS_PER_WORKER

        gather_h = [None] * _NBUF
        out_h = [None] * _NBUF
        offs = [None] * _NBUF

        def retire(k):
            kbuf = k % _NBUF
            gather_h[kbuf].wait()
            out_h[kbuf] = pltpu.async_copy(
                rows[kbuf],
                out_hbm.at[pl.ds(offs[kbuf], _CHUNK)],
                osem[kbuf],
            )

        for k in range(_UNITS_PER_WORKER):
            buf = k % _NBUF
            u = u0 + k
            t = u // _BLOCKS_PER_T
            b0 = (u % _BLOCKS_PER_T) * _CHUNK
            if out_h[buf] is not None:
                out_h[buf].wait()
            offs[buf] = t * _BATCH + b0
            pltpu.sync_copy(idx_hbm.at[t, pl.ds(b0, _CHUNK)], idx_v[buf])
            gather_h[buf] = pltpu.async_copy(
                table_hbm.at[idx_v[buf]], rows[buf], gsem[buf]
            )
            if k >= _NBUF - 1:
                retire(k - (_NBUF - 1))

        for k in range(_UNITS_PER_WORKER - (_NBUF - 1), _UNITS_PER_WORKER):
            retire(k)
        for h in out_h:
            if h is not None:
                h.wait()

    out2d = gather_kernel(rt, idx2)
    out_tdb = _unpack_out(out2d.reshape(_NUM_IDX // 4, 128))
    return out_tdb.transpose(2, 0, 1)


# speedup vs baseline: 1.1602x; 1.1602x over previous
import functools

import jax
import jax.numpy as jnp
from jax import lax
from jax.experimental import pallas as pl
from jax.experimental.pallas import tpu as pltpu
from jax.experimental.pallas import tpu_sc as plsc

_BATCH = 16384
_HIST = 50
_DIM = 32
_VOCAB_PAD = 1 << 20
_NUM_IDX = _BATCH * _HIST
_NUM_WORKERS = 32
_CHUNK = 1024
_BLOCKS_PER_T = _BATCH // _CHUNK
_NUM_UNITS = _HIST * _BLOCKS_PER_T
_UNITS_PER_WORKER = _NUM_UNITS // _NUM_WORKERS
_NBUF = 3

_PACK_B = 2048
_PACK_Q = _PACK_B // 4
_PACK_GRID = 489

_UNPACK_Q = _CHUNK // 4


def _pack_table(tt):
    def body(tt_ref, o_ref):
        t = tt_ref[...]
        c = jnp.concatenate(
            [t[:, _PACK_Q * s : _PACK_Q * (s + 1)] for s in range(4)], axis=0
        )
        o_ref[...] = c.T

    return pl.pallas_call(
        body,
        grid=(_PACK_GRID,),
        in_specs=[pl.BlockSpec((_DIM, _PACK_B), lambda i: (0, i))],
        out_specs=pl.BlockSpec((_PACK_Q, 128), lambda i: (i, 0)),
        out_shape=jax.ShapeDtypeStruct((_VOCAB_PAD // 4, 128), jnp.float32),
    )(tt)


def _unpack_out(packed):
    def body(in_ref, o_ref):
        rt_ = in_ref[...].T
        for s in range(4):
            o_ref[0, :, _UNPACK_Q * s : _UNPACK_Q * (s + 1)] = rt_[
                32 * s : 32 * (s + 1), :
            ]

    return pl.pallas_call(
        body,
        grid=(_HIST, _BLOCKS_PER_T),
        in_specs=[
            pl.BlockSpec(
                (_CHUNK // 4, 128),
                lambda t, j: (t * _BLOCKS_PER_T + j, 0),
            )
        ],
        out_specs=pl.BlockSpec((1, _DIM, _CHUNK), lambda t, j: (t, 0, j)),
        out_shape=jax.ShapeDtypeStruct((_HIST, _DIM, _BATCH), jnp.float32),
    )(packed)


def kernel(x, table):
    xt = x.T
    xt_p = (
        xt.reshape(_HIST, _BLOCKS_PER_T, 4, _UNPACK_Q)
        .transpose(0, 1, 3, 2)
        .reshape(_HIST, _BATCH)
    )
    idx2 = (xt_p & -2048) | ((xt_p & 511) << 2) | ((xt_p >> 9) & 3)

    rt = _pack_table(table.T).reshape(_VOCAB_PAD, _DIM)
    mesh = plsc.VectorSubcoreMesh(core_axis_name="c", subcore_axis_name="s")

    @functools.partial(
        pl.kernel,
        mesh=mesh,
        out_type=jax.ShapeDtypeStruct((_NUM_IDX, _DIM), jnp.float32),
        compiler_params=pltpu.CompilerParams(use_tc_tiling_on_sc=False),
        scratch_types=(
            [pltpu.VMEM((_CHUNK,), jnp.int32)] * _NBUF
            + [pltpu.VMEM((_CHUNK, _DIM), jnp.float32)] * _NBUF
            + [pltpu.SemaphoreType.DMA] * (2 * _NBUF)
        ),
    )
    def gather_kernel(table_hbm, idx_hbm, out_hbm, *scratch):
        idx_v = scratch[:_NBUF]
        rows = scratch[_NBUF : 2 * _NBUF]
        gsem = scratch[2 * _NBUF : 3 * _NBUF]
        osem = scratch[3 * _NBUF :]
        wid = lax.axis_index("s") * 2 + lax.axis_index("c")
        u0 = wid * _UNITS_PER_WORKER

        gather_h = [None] * _NBUF
        out_h = [None] * _NBUF
        offs = [None] * _NBUF

        def retire(k):
            kbuf = k % _NBUF
            gather_h[kbuf].wait()
            out_h[kbuf] = pltpu.async_copy(
                rows[kbuf],
                out_hbm.at[pl.ds(offs[kbuf], _CHUNK)],
                osem[kbuf],
            )

        for k in range(_UNITS_PER_WORKER):
            buf = k % _NBUF
            u = u0 + k
            t = u // _BLOCKS_PER_T
            b0 = (u % _BLOCKS_PER_T) * _CHUNK
            if out_h[buf] is not None:
                out_h[buf].wait()
            offs[buf] = t * _BATCH + b0
            pltpu.sync_copy(idx_hbm.at[t, pl.ds(b0, _CHUNK)], idx_v[buf])
            gather_h[buf] = pltpu.async_copy(
                table_hbm.at[idx_v[buf]], rows[buf], gsem[buf]
            )
            if k >= _NBUF - 1:
                retire(k - (_NBUF - 1))

        for k in range(_UNITS_PER_WORKER - (_NBUF - 1), _UNITS_PER_WORKER):
            retire(k)
        for h in out_h:
            if h is not None:
                h.wait()

    out2d = gather_kernel(rt, idx2)
    out_tdb = _unpack_out(out2d.reshape(_NUM_IDX // 4, 128))
    return out_tdb.transpose(2, 0, 1)


# speedup vs baseline: 1.1606x; 1.0003x over previous
import functools

import jax
import jax.numpy as jnp
from jax import lax
from jax.experimental import pallas as pl
from jax.experimental.pallas import tpu as pltpu
from jax.experimental.pallas import tpu_sc as plsc

_BATCH = 16384
_HIST = 50
_DIM = 32
_VOCAB_PAD = 1 << 20
_NUM_IDX = _BATCH * _HIST
_NUM_WORKERS = 32
_CHUNK = 1024
_BLOCKS_PER_T = _BATCH // _CHUNK
_NUM_UNITS = _HIST * _BLOCKS_PER_T
_UNITS_PER_WORKER = _NUM_UNITS // _NUM_WORKERS
_NBUF = 3

_PACK_B = 2048
_PACK_Q = _PACK_B // 4
_PACK_GRID = 489

_UNPACK_Q = _CHUNK // 4


def _pack_table(tt):
    def body(tt_ref, o_ref):
        t = tt_ref[...]
        c = jnp.concatenate(
            [t[:, _PACK_Q * s : _PACK_Q * (s + 1)] for s in range(4)], axis=0
        )
        o_ref[...] = c.T

    return pl.pallas_call(
        body,
        grid=(_PACK_GRID,),
        in_specs=[pl.BlockSpec((_DIM, _PACK_B), lambda i: (0, i))],
        out_specs=pl.BlockSpec((_PACK_Q, 128), lambda i: (i, 0)),
        out_shape=jax.ShapeDtypeStruct((_VOCAB_PAD // 4, 128), jnp.float32),
        compiler_params=pltpu.CompilerParams(dimension_semantics=("parallel",)),
    )(tt)


def _unpack_out(packed):
    def body(in_ref, o_ref):
        rt_ = in_ref[...].T
        for s in range(4):
            o_ref[0, :, _UNPACK_Q * s : _UNPACK_Q * (s + 1)] = rt_[
                32 * s : 32 * (s + 1), :
            ]

    return pl.pallas_call(
        body,
        grid=(_HIST, _BLOCKS_PER_T),
        in_specs=[
            pl.BlockSpec(
                (_CHUNK // 4, 128),
                lambda t, j: (t * _BLOCKS_PER_T + j, 0),
            )
        ],
        out_specs=pl.BlockSpec((1, _DIM, _CHUNK), lambda t, j: (t, 0, j)),
        out_shape=jax.ShapeDtypeStruct((_HIST, _DIM, _BATCH), jnp.float32),
        compiler_params=pltpu.CompilerParams(
            dimension_semantics=("parallel", "parallel")
        ),
    )(packed)


def kernel(x, table):
    xt = x.T
    xt_p = (
        xt.reshape(_HIST, _BLOCKS_PER_T, 4, _UNPACK_Q)
        .transpose(0, 1, 3, 2)
        .reshape(_HIST, _BATCH)
    )
    idx2 = (xt_p & -2048) | ((xt_p & 511) << 2) | ((xt_p >> 9) & 3)

    rt = _pack_table(table.T).reshape(_VOCAB_PAD, _DIM)
    mesh = plsc.VectorSubcoreMesh(core_axis_name="c", subcore_axis_name="s")

    @functools.partial(
        pl.kernel,
        mesh=mesh,
        out_type=jax.ShapeDtypeStruct((_NUM_IDX, _DIM), jnp.float32),
        compiler_params=pltpu.CompilerParams(use_tc_tiling_on_sc=False),
        scratch_types=(
            [pltpu.VMEM((_CHUNK,), jnp.int32)] * _NBUF
            + [pltpu.VMEM((_CHUNK, _DIM), jnp.float32)] * _NBUF
            + [pltpu.SemaphoreType.DMA] * (2 * _NBUF)
        ),
    )
    def gather_kernel(table_hbm, idx_hbm, out_hbm, *scratch):
        idx_v = scratch[:_NBUF]
        rows = scratch[_NBUF : 2 * _NBUF]
        gsem = scratch[2 * _NBUF : 3 * _NBUF]
        osem = scratch[3 * _NBUF :]
        wid = lax.axis_index("s") * 2 + lax.axis_index("c")
        u0 = wid * _UNITS_PER_WORKER

        gather_h = [None] * _NBUF
        out_h = [None] * _NBUF
        offs = [None] * _NBUF

        def retire(k):
            kbuf = k % _NBUF
            gather_h[kbuf].wait()
            out_h[kbuf] = pltpu.async_copy(
                rows[kbuf],
                out_hbm.at[pl.ds(offs[kbuf], _CHUNK)],
                osem[kbuf],
            )

        for k in range(_UNITS_PER_WORKER):
            buf = k % _NBUF
            u = u0 + k
            t = u // _BLOCKS_PER_T
            b0 = (u % _BLOCKS_PER_T) * _CHUNK
            if out_h[buf] is not None:
                out_h[buf].wait()
            offs[buf] = t * _BATCH + b0
            pltpu.sync_copy(idx_hbm.at[t, pl.ds(b0, _CHUNK)], idx_v[buf])
            gather_h[buf] = pltpu.async_copy(
                table_hbm.at[idx_v[buf]], rows[buf], gsem[buf]
            )
            if k >= _NBUF - 1:
                retire(k - (_NBUF - 1))

        for k in range(_UNITS_PER_WORKER - (_NBUF - 1), _UNITS_PER_WORKER):
            retire(k)
        for h in out_h:
            if h is not None:
                h.wait()

    out2d = gather_kernel(rt, idx2)
    out_tdb = _unpack_out(out2d.reshape(_NUM_IDX // 4, 128))
    return out_tdb.transpose(2, 0, 1)
